# baseline (speedup 1.0000x reference)
import functools

import jax
import jax.numpy as jnp
from jax import lax
from jax.experimental import pallas as pl
from jax.experimental.pallas import tpu as pltpu
from jax.experimental.pallas import tpu_sc as plsc

N = 65536
S = 1024
K1, K2 = 32, 64
R1SQ, R2SQ = 0.1 * 0.1, 0.2 * 0.2
EPS = 1e-5

NRES = 24576
CHUNK = 4096
NWORKERS = 32
QPW = S // NWORKERS


def _scan_step(xs, ys, zs, x2s, base_idx, v, qx, qy, qz, q2, s1, s2,
               idx1buf, idx2buf):
    d2 = (q2 + x2s) - 2.0 * ((qx * xs + qy * ys) + qz * zs)
    gi = lax.iota(jnp.int32, 16) + (base_idx + v * 16)
    m1 = (d2 < R1SQ) & jnp.full((16,), s1 < K1)
    plsc.store_compressed(idx1buf.at[pl.ds(s1, 16)], gi, mask=m1)
    s1 = s1 + plsc.all_reduce_population_count(m1)[0]
    m2 = (d2 < R2SQ) & jnp.full((16,), s2 < K2)
    plsc.store_compressed(idx2buf.at[pl.ds(s2, 16)], gi, mask=m2)
    s2 = s2 + plsc.all_reduce_population_count(m2)[0]
    return s1, s2


def _make_sc_ballquery():
    mesh = plsc.VectorSubcoreMesh(core_axis_name="c", subcore_axis_name="s")

    @functools.partial(
        pl.kernel,
        mesh=mesh,
        compiler_params=pltpu.CompilerParams(
            needs_layout_passes=False, use_tc_tiling_on_sc=False),
        out_type=[
            jax.ShapeDtypeStruct((S * K1, 16), jnp.float32),
            jax.ShapeDtypeStruct((S * K2, 16), jnp.float32),
        ],
        scratch_types=[
            pltpu.VMEM((NRES,), jnp.float32),
            pltpu.VMEM((NRES,), jnp.float32),
            pltpu.VMEM((NRES,), jnp.float32),
            pltpu.VMEM((NRES,), jnp.float32),
            pltpu.VMEM((CHUNK,), jnp.float32),
            pltpu.VMEM((CHUNK,), jnp.float32),
            pltpu.VMEM((CHUNK,), jnp.float32),
            pltpu.VMEM((CHUNK,), jnp.float32),
            pltpu.VMEM((QPW,), jnp.float32),
            pltpu.VMEM((QPW,), jnp.float32),
            pltpu.VMEM((QPW,), jnp.float32),
            pltpu.VMEM((QPW,), jnp.float32),
            pltpu.VMEM((304,), jnp.int32),
            pltpu.VMEM((336,), jnp.int32),
            pltpu.VMEM((2, K1), jnp.int32),
            pltpu.VMEM((2, K2), jnp.int32),
            pltpu.VMEM((2, K1, 16), jnp.float32),
            pltpu.VMEM((2, K2, 16), jnp.float32),
            pltpu.SemaphoreType.DMA,
            pltpu.SemaphoreType.DMA,
        ],
    )
    def sc_ballquery(ptsx, ptsy, ptsz, ptsx2, ptspad, qxh, qyh, qzh, q2h,
                     g1_hbm, g2_hbm,
                     resx, resy, resz, resx2, chx, chy, chz, chx2,
                     qxb, qyb, qzb, q2b,
                     idx1buf, idx2buf, idx1sel, idx2sel, rows1, rows2,
                     semG, semW):
        wid = lax.axis_index("s") * 2 + lax.axis_index("c")
        qbase = wid * QPW
        pltpu.sync_copy(ptsx.at[pl.ds(0, NRES)], resx)
        pltpu.sync_copy(ptsy.at[pl.ds(0, NRES)], resy)
        pltpu.sync_copy(ptsz.at[pl.ds(0, NRES)], resz)
        pltpu.sync_copy(ptsx2.at[pl.ds(0, NRES)], resx2)
        pltpu.sync_copy(qxh.at[pl.ds(qbase, QPW)], qxb)
        pltpu.sync_copy(qyh.at[pl.ds(qbase, QPW)], qyb)
        pltpu.sync_copy(qzh.at[pl.ds(qbase, QPW)], qzb)
        pltpu.sync_copy(q2h.at[pl.ds(qbase, QPW)], q2b)

        def per_query(i, carry):
            qid = qbase + i
            ii = jnp.full((16,), i, jnp.int32)
            qx = plsc.load_gather(qxb, [ii])
            qy = plsc.load_gather(qyb, [ii])
            qz = plsc.load_gather(qzb, [ii])
            q2 = plsc.load_gather(q2b, [ii])

            def cond_a(c):
                v, s1, s2 = c
                return ((s1 < K1) | (s2 < K2)) & (v < NRES // 256)

            def body_a(c):
                v, s1, s2 = c
                d2s = []
                gis = []
                for u in range(16):
                    vv = v * 16 + u
                    xs = resx[pl.ds(vv * 16, 16)]
                    ys = resy[pl.ds(vv * 16, 16)]
                    zs = resz[pl.ds(vv * 16, 16)]
                    x2s = resx2[pl.ds(vv * 16, 16)]
                    d2s.append((q2 + x2s) - 2.0 * ((qx * xs + qy * ys)
                                                   + qz * zs))
                    gis.append(lax.iota(jnp.int32, 16) + vv * 16)
                c1 = jnp.full((16,), s1 < K1)
                c2 = jnp.full((16,), s2 < K2)
                m1s = [(d2 < R1SQ) & c1 for d2 in d2s]
                m2s = [(d2 < R2SQ) & c2 for d2 in d2s]

                def count(ms):
                    a = (ms[0] | ms[1]) | (ms[2] | ms[3])
                    b = (ms[4] | ms[5]) | (ms[6] | ms[7])
                    c = (ms[8] | ms[9]) | (ms[10] | ms[11])
                    d = (ms[12] | ms[13]) | (ms[14] | ms[15])
                    anym = (a | b) | (c | d)
                    return plsc.all_reduce_population_count(anym)[0]

                def slow1(s):
                    for u in range(16):
                        plsc.store_compressed(idx1buf.at[pl.ds(s, 16)],
                                              gis[u], mask=m1s[u])
                        s = s + plsc.all_reduce_population_count(m1s[u])[0]
                    return s

                def slow2(s):
                    for u in range(16):
                        plsc.store_compressed(idx2buf.at[pl.ds(s, 16)],
                                              gis[u], mask=m2s[u])
                        s = s + plsc.all_reduce_population_count(m2s[u])[0]
                    return s

                s1 = lax.cond(count(m1s) > 0, slow1, lambda s: s, s1)
                s2 = lax.cond(count(m2s) > 0, slow2, lambda s: s, s2)
                return v + 1, s1, s2

            _, s1, s2 = lax.while_loop(
                cond_a, body_a, (jnp.int32(0), jnp.int32(0), jnp.int32(0)))

            def cond_b(c):
                ch, s1, s2 = c
                return ((s1 < K1) | (s2 < K2)) & (ch < N // CHUNK)

            def body_b(c):
                ch, s1, s2 = c
                base = ch * CHUNK
                pltpu.sync_copy(ptsx.at[pl.ds(base, CHUNK)], chx)
                pltpu.sync_copy(ptsy.at[pl.ds(base, CHUNK)], chy)
                pltpu.sync_copy(ptsz.at[pl.ds(base, CHUNK)], chz)
                pltpu.sync_copy(ptsx2.at[pl.ds(base, CHUNK)], chx2)

                def cond_i(c2):
                    v, s1, s2 = c2
                    return ((s1 < K1) | (s2 < K2)) & (v < CHUNK // 16)

                def body_i(c2):
                    v, s1, s2 = c2
                    xs = chx[pl.ds(v * 16, 16)]
                    ys = chy[pl.ds(v * 16, 16)]
                    zs = chz[pl.ds(v * 16, 16)]
                    x2s = chx2[pl.ds(v * 16, 16)]
                    s1, s2 = _scan_step(xs, ys, zs, x2s, base, v, qx, qy, qz,
                                        q2, s1, s2, idx1buf, idx2buf)
                    return v + 1, s1, s2

                _, s1, s2 = lax.while_loop(cond_i, body_i,
                                           (jnp.int32(0), s1, s2))
                return ch + 1, s1, s2

            _, s1, s2 = lax.while_loop(cond_b, body_b,
                                       (jnp.int32(NRES // CHUNK), s1, s2))

            p = lax.rem(i, 2)
            zero16 = jnp.zeros((16,), jnp.int32)
            lanes = lax.iota(jnp.int32, 16)
            s1c = jnp.minimum(s1, K1)
            first1 = plsc.load_gather(idx1buf, [zero16])
            for j in range(K1 // 16):
                vals = idx1buf[pl.ds(j * 16, 16)]
                sel = jnp.where(lanes + j * 16 < jnp.full((16,), s1c),
                                vals, first1)
                idx1sel[p, pl.ds(j * 16, 16)] = sel
            s2c = jnp.minimum(s2, K2)
            first2 = plsc.load_gather(idx2buf, [zero16])
            for j in range(K2 // 16):
                vals = idx2buf[pl.ds(j * 16, 16)]
                sel = jnp.where(lanes + j * 16 < jnp.full((16,), s2c),
                                vals, first2)
                idx2sel[p, pl.ds(j * 16, 16)] = sel

            @pl.when(i > 0)
            def _drain_gathers():
                pltpu.make_async_copy(
                    ptspad.at[pl.ds(0, K1)], rows1.at[0], semG).wait()
                pltpu.make_async_copy(
                    ptspad.at[pl.ds(0, K2)], rows2.at[0], semG).wait()

            @pl.when(i > 1)
            def _drain_writes():
                pltpu.make_async_copy(
                    ptspad.at[pl.ds(0, K1)], rows1.at[0], semW).wait()
                pltpu.make_async_copy(
                    ptspad.at[pl.ds(0, K2)], rows2.at[0], semW).wait()

            @pl.when(i > 0)
            def _write_prev():
                pltpu.async_copy(rows1.at[1 - p],
                                 g1_hbm.at[pl.ds((qid - 1) * K1, K1)], semW)
                pltpu.async_copy(rows2.at[1 - p],
                                 g2_hbm.at[pl.ds((qid - 1) * K2, K2)], semW)

            pltpu.async_copy(ptspad.at[idx1sel.at[p]], rows1.at[p], semG)
            pltpu.async_copy(ptspad.at[idx2sel.at[p]], rows2.at[p], semG)
            return carry

        lax.fori_loop(0, QPW, per_query, jnp.int32(0))

        pltpu.make_async_copy(ptspad.at[pl.ds(0, K1)], rows1.at[0], semG).wait()
        pltpu.make_async_copy(ptspad.at[pl.ds(0, K2)], rows2.at[0], semG).wait()
        pltpu.make_async_copy(ptspad.at[pl.ds(0, K1)], rows1.at[0], semW).wait()
        pltpu.make_async_copy(ptspad.at[pl.ds(0, K2)], rows2.at[0], semW).wait()
        qlast = qbase + QPW - 1
        pltpu.sync_copy(rows1.at[(QPW - 1) % 2],
                        g1_hbm.at[pl.ds(qlast * K1, K1)])
        pltpu.sync_copy(rows2.at[(QPW - 1) % 2],
                        g2_hbm.at[pl.ds(qlast * K2, K2)])

    return sc_ballquery


_SC_CACHE = []


def _sc_ballquery(*args):
    if not _SC_CACHE:
        _SC_CACHE.append(_make_sc_ballquery())
    return _SC_CACHE[0](*args)


def _mlp_body(g1_ref, g2_ref, q1_ref, q2_ref, w1a_ref, b1a_ref, w1b_ref,
              b1b_ref, w2a_ref, b2a_ref, w2b_ref, b2b_ref, w3a_ref, w3b_ref,
              b3_ref, out_ref):
    qblk = out_ref.shape[0]
    r_i = lax.broadcasted_iota(jnp.int32, (128, 128), 0)
    c_i = lax.broadcasted_iota(jnp.int32, (128, 128), 1)
    shift = (((c_i // 16) == (r_i // 16))
             & ((c_i % 16) == (r_i % 16) + 3)
             & ((r_i % 16) < 3)).astype(jnp.float32)

    def branch(g_ref, q_ref, k, wa_ref, ba_ref, wb_ref, bb_ref):
        g = g_ref[...]
        f = (g - q_ref[...]) + jnp.dot(g, shift, precision="highest")
        h = jnp.dot(f, wa_ref[...], precision="highest") + ba_ref[0:1, :]
        h = jnp.maximum(h, 0.0)
        h = jnp.dot(h, wb_ref[...], precision="highest") + bb_ref[0:1, :]
        h = jnp.maximum(h, 0.0)
        h = jnp.max(h.reshape(qblk, k // 8, 256), axis=1)
        m = jnp.maximum(h[:, :128], h[:, 128:])
        m = jnp.maximum(m, pltpu.roll(m, 64, 1))
        m = jnp.maximum(m, pltpu.roll(m, 32, 1))
        return m[:, :32]

    f1 = branch(g1_ref, q1_ref, K1, w1a_ref, b1a_ref, w1b_ref, b1b_ref)
    f2 = branch(g2_ref, q2_ref, K2, w2a_ref, b2a_ref, w2b_ref, b2b_ref)
    out = (jnp.dot(f1, w3a_ref[...], precision="highest")
           + jnp.dot(f2, w3b_ref[...], precision="highest") + b3_ref[0:1, :])
    out_ref[...] = out


def _run_mlp(g1p, g2p, q1p, q2p, w1a, b1a, w1b, b1b, w2a, b2a, w2b, b2b,
             w3a, w3b, b3):
    qblk = 128
    grid = (S // qblk,)

    def full(a):
        return pl.BlockSpec(a.shape, lambda i: (0, 0))

    return pl.pallas_call(
        _mlp_body,
        grid=grid,
        in_specs=[
            pl.BlockSpec((qblk * K1 // 8, 128), lambda i: (i, 0)),
            pl.BlockSpec((qblk * K2 // 8, 128), lambda i: (i, 0)),
            pl.BlockSpec((qblk * K1 // 8, 128), lambda i: (i, 0)),
            pl.BlockSpec((qblk * K2 // 8, 128), lambda i: (i, 0)),
            full(w1a), full(b1a), full(w1b), full(b1b),
            full(w2a), full(b2a), full(w2b), full(b2b),
            full(w3a), full(w3b), full(b3),
        ],
        out_specs=pl.BlockSpec((qblk, 128), lambda i: (i, 0)),
        out_shape=jax.ShapeDtypeStruct((S, 128), jnp.float32),
    )(g1p, g2p, q1p, q2p, w1a, b1a, w1b, b1b, w2a, b2a, w2b, b2b,
      w3a, w3b, b3)


def kernel(pts1, W1a, g1a, b1a, W1b, g1b, b1b, W2a, g2a, b2a, W2b, g2b, b2b,
           W3, b3):
    b = pts1.shape[0]
    p = pts1[0]
    grid = p.reshape(256, 256, 3)
    pts2 = grid[4::8, 4::8].reshape(-1, 3)

    def bf16_rne(x):
        u = lax.bitcast_convert_type(x, jnp.uint32)
        r = (u + jnp.uint32(0x7FFF) + ((u >> 16) & jnp.uint32(1)))
        r = r & jnp.uint32(0xFFFF0000)
        return lax.bitcast_convert_type(r, jnp.float32)

    pbf = bf16_rne(p)
    x2 = jnp.sum(p * p, axis=-1)
    qbf = bf16_rne(pts2)
    q2 = jnp.sum(pts2 * pts2, axis=-1)
    ptspad = jnp.pad(p, ((0, 0), (0, 13)))
    qpad = jnp.pad(pts2, ((0, 0), (0, 13)))

    g1, g2 = _sc_ballquery(pbf[:, 0], pbf[:, 1], pbf[:, 2], x2, ptspad,
                           qbf[:, 0], qbf[:, 1], qbf[:, 2], q2)
    g1p = g1.reshape(S * K1 // 8, 128)
    g2p = g2.reshape(S * K2 // 8, 128)
    q1p = jnp.broadcast_to(qpad[:, None, :], (S, K1, 16)).reshape(-1, 128)
    q2p = jnp.broadcast_to(qpad[:, None, :], (S, K2, 16)).reshape(-1, 128)

    scale = 1.0 / jnp.sqrt(1.0 + EPS)
    eye8 = jnp.eye(8, dtype=jnp.float32)

    def tile8(bias):
        return jnp.tile(bias[None, :], (8, 8 * 16 // bias.shape[0]))

    def fold(w, g, bias, rows):
        wf = w * (g * scale)[None, :]
        wpad = jnp.zeros((16, w.shape[1]), w.dtype).at[:rows].set(wf)
        return jnp.kron(eye8, wpad), tile8(bias)

    w1a_p, b1a_p = fold(W1a, g1a, b1a, 6)
    w2a_p, b2a_p = fold(W2a, g2a, b2a, 6)
    w1b_f = jnp.kron(eye8, W1b * (g1b * scale)[None, :])
    w2b_f = jnp.kron(eye8, W2b * (g2b * scale)[None, :])

    def tile8b(bias):
        return jnp.tile(bias[None, :], (8, 8))

    out = _run_mlp(g1p, g2p, q1p, q2p,
                   w1a_p, b1a_p, w1b_f, tile8b(b1b),
                   w2a_p, b2a_p, w2b_f, tile8b(b2b),
                   W3[:32], W3[32:], jnp.tile(b3[None, :], (8, 1)))
    return out.reshape(b, S, 128)


# speedup vs baseline: 1.0199x; 1.0199x over previous
import functools

import jax
import jax.numpy as jnp
from jax import lax
from jax.experimental import pallas as pl
from jax.experimental.pallas import tpu as pltpu
from jax.experimental.pallas import tpu_sc as plsc

N = 65536
S = 1024
K1, K2 = 32, 64
R1SQ, R2SQ = 0.1 * 0.1, 0.2 * 0.2
EPS = 1e-5

NRES = 24576
CHUNK = 4096
NWORKERS = 32
QPW = S // NWORKERS


def _scan_step(xs, ys, zs, x2s, base_idx, v, qx, qy, qz, q2, s1, s2,
               idx1buf, idx2buf):
    d2 = (q2 + x2s) - 2.0 * ((qx * xs + qy * ys) + qz * zs)
    gi = lax.iota(jnp.int32, 16) + (base_idx + v * 16)
    m1 = (d2 < R1SQ) & jnp.full((16,), s1 < K1)
    plsc.store_compressed(idx1buf.at[pl.ds(s1, 16)], gi, mask=m1)
    s1 = s1 + plsc.all_reduce_population_count(m1)[0]
    m2 = (d2 < R2SQ) & jnp.full((16,), s2 < K2)
    plsc.store_compressed(idx2buf.at[pl.ds(s2, 16)], gi, mask=m2)
    s2 = s2 + plsc.all_reduce_population_count(m2)[0]
    return s1, s2


def _make_sc_ballquery():
    mesh = plsc.VectorSubcoreMesh(core_axis_name="c", subcore_axis_name="s")

    @functools.partial(
        pl.kernel,
        mesh=mesh,
        compiler_params=pltpu.CompilerParams(
            needs_layout_passes=False, use_tc_tiling_on_sc=False),
        out_type=[
            jax.ShapeDtypeStruct((S * K1, 16), jnp.float32),
            jax.ShapeDtypeStruct((S * K2, 16), jnp.float32),
        ],
        scratch_types=[
            pltpu.VMEM((NRES,), jnp.float32),
            pltpu.VMEM((NRES,), jnp.float32),
            pltpu.VMEM((NRES,), jnp.float32),
            pltpu.VMEM((NRES,), jnp.float32),
            pltpu.VMEM((CHUNK,), jnp.float32),
            pltpu.VMEM((CHUNK,), jnp.float32),
            pltpu.VMEM((CHUNK,), jnp.float32),
            pltpu.VMEM((CHUNK,), jnp.float32),
            pltpu.VMEM((QPW,), jnp.float32),
            pltpu.VMEM((QPW,), jnp.float32),
            pltpu.VMEM((QPW,), jnp.float32),
            pltpu.VMEM((QPW,), jnp.float32),
            pltpu.VMEM((176,), jnp.int32),
            pltpu.VMEM((208,), jnp.int32),
            pltpu.VMEM((2, K1), jnp.int32),
            pltpu.VMEM((2, K2), jnp.int32),
            pltpu.VMEM((2, K1, 16), jnp.float32),
            pltpu.VMEM((2, K2, 16), jnp.float32),
            pltpu.SemaphoreType.DMA,
            pltpu.SemaphoreType.DMA,
        ],
    )
    def sc_ballquery(ptsx, ptsy, ptsz, ptsx2, ptspad, qxh, qyh, qzh, q2h,
                     g1_hbm, g2_hbm,
                     resx, resy, resz, resx2, chx, chy, chz, chx2,
                     qxb, qyb, qzb, q2b,
                     idx1buf, idx2buf, idx1sel, idx2sel, rows1, rows2,
                     semG, semW):
        wid = lax.axis_index("s") * 2 + lax.axis_index("c")
        qbase = wid * QPW
        pltpu.sync_copy(ptsx.at[pl.ds(0, NRES)], resx)
        pltpu.sync_copy(ptsy.at[pl.ds(0, NRES)], resy)
        pltpu.sync_copy(ptsz.at[pl.ds(0, NRES)], resz)
        pltpu.sync_copy(ptsx2.at[pl.ds(0, NRES)], resx2)
        pltpu.sync_copy(qxh.at[pl.ds(qbase, QPW)], qxb)
        pltpu.sync_copy(qyh.at[pl.ds(qbase, QPW)], qyb)
        pltpu.sync_copy(qzh.at[pl.ds(qbase, QPW)], qzb)
        pltpu.sync_copy(q2h.at[pl.ds(qbase, QPW)], q2b)

        def per_query(i, carry):
            qid = qbase + i
            ii = jnp.full((16,), i, jnp.int32)
            qx = plsc.load_gather(qxb, [ii])
            qy = plsc.load_gather(qyb, [ii])
            qz = plsc.load_gather(qzb, [ii])
            q2 = plsc.load_gather(q2b, [ii])

            def cond_a(c):
                v, s1, s2 = c
                return ((s1 < K1) | (s2 < K2)) & (v < NRES // 128)

            def body_a(c):
                v, s1, s2 = c
                d2s = []
                gis = []
                for u in range(8):
                    vv = v * 8 + u
                    xs = resx[pl.ds(vv * 16, 16)]
                    ys = resy[pl.ds(vv * 16, 16)]
                    zs = resz[pl.ds(vv * 16, 16)]
                    x2s = resx2[pl.ds(vv * 16, 16)]
                    d2s.append((q2 + x2s) - 2.0 * ((qx * xs + qy * ys)
                                                   + qz * zs))
                    gis.append(lax.iota(jnp.int32, 16) + vv * 16)
                c1 = jnp.full((16,), s1 < K1)
                c2 = jnp.full((16,), s2 < K2)
                m1s = [(d2 < R1SQ) & c1 for d2 in d2s]
                m2s = [(d2 < R2SQ) & c2 for d2 in d2s]

                def count(ms):
                    anym = ((ms[0] | ms[1]) | (ms[2] | ms[3])) \
                        | ((ms[4] | ms[5]) | (ms[6] | ms[7]))
                    return plsc.all_reduce_population_count(anym)[0]

                def slow1(s):
                    for u in range(8):
                        plsc.store_compressed(idx1buf.at[pl.ds(s, 16)],
                                              gis[u], mask=m1s[u])
                        s = s + plsc.all_reduce_population_count(m1s[u])[0]
                    return s

                def slow2(s):
                    for u in range(8):
                        plsc.store_compressed(idx2buf.at[pl.ds(s, 16)],
                                              gis[u], mask=m2s[u])
                        s = s + plsc.all_reduce_population_count(m2s[u])[0]
                    return s

                s1 = lax.cond(count(m1s) > 0, slow1, lambda s: s, s1)
                s2 = lax.cond(count(m2s) > 0, slow2, lambda s: s, s2)
                return v + 1, s1, s2

            _, s1, s2 = lax.while_loop(
                cond_a, body_a, (jnp.int32(0), jnp.int32(0), jnp.int32(0)))

            def cond_b(c):
                ch, s1, s2 = c
                return ((s1 < K1) | (s2 < K2)) & (ch < N // CHUNK)

            def body_b(c):
                ch, s1, s2 = c
                base = ch * CHUNK
                pltpu.sync_copy(ptsx.at[pl.ds(base, CHUNK)], chx)
                pltpu.sync_copy(ptsy.at[pl.ds(base, CHUNK)], chy)
                pltpu.sync_copy(ptsz.at[pl.ds(base, CHUNK)], chz)
                pltpu.sync_copy(ptsx2.at[pl.ds(base, CHUNK)], chx2)

                def cond_i(c2):
                    v, s1, s2 = c2
                    return ((s1 < K1) | (s2 < K2)) & (v < CHUNK // 16)

                def body_i(c2):
                    v, s1, s2 = c2
                    xs = chx[pl.ds(v * 16, 16)]
                    ys = chy[pl.ds(v * 16, 16)]
                    zs = chz[pl.ds(v * 16, 16)]
                    x2s = chx2[pl.ds(v * 16, 16)]
                    s1, s2 = _scan_step(xs, ys, zs, x2s, base, v, qx, qy, qz,
                                        q2, s1, s2, idx1buf, idx2buf)
                    return v + 1, s1, s2

                _, s1, s2 = lax.while_loop(cond_i, body_i,
                                           (jnp.int32(0), s1, s2))
                return ch + 1, s1, s2

            _, s1, s2 = lax.while_loop(cond_b, body_b,
                                       (jnp.int32(NRES // CHUNK), s1, s2))

            p = lax.rem(i, 2)
            zero16 = jnp.zeros((16,), jnp.int32)
            lanes = lax.iota(jnp.int32, 16)
            s1c = jnp.minimum(s1, K1)
            first1 = plsc.load_gather(idx1buf, [zero16])
            for j in range(K1 // 16):
                vals = idx1buf[pl.ds(j * 16, 16)]
                sel = jnp.where(lanes + j * 16 < jnp.full((16,), s1c),
                                vals, first1)
                idx1sel[p, pl.ds(j * 16, 16)] = sel
            s2c = jnp.minimum(s2, K2)
            first2 = plsc.load_gather(idx2buf, [zero16])
            for j in range(K2 // 16):
                vals = idx2buf[pl.ds(j * 16, 16)]
                sel = jnp.where(lanes + j * 16 < jnp.full((16,), s2c),
                                vals, first2)
                idx2sel[p, pl.ds(j * 16, 16)] = sel

            @pl.when(i > 0)
            def _drain_gathers():
                pltpu.make_async_copy(
                    ptspad.at[pl.ds(0, K1)], rows1.at[0], semG).wait()
                pltpu.make_async_copy(
                    ptspad.at[pl.ds(0, K2)], rows2.at[0], semG).wait()

            @pl.when(i > 1)
            def _drain_writes():
                pltpu.make_async_copy(
                    ptspad.at[pl.ds(0, K1)], rows1.at[0], semW).wait()
                pltpu.make_async_copy(
                    ptspad.at[pl.ds(0, K2)], rows2.at[0], semW).wait()

            @pl.when(i > 0)
            def _write_prev():
                pltpu.async_copy(rows1.at[1 - p],
                                 g1_hbm.at[pl.ds((qid - 1) * K1, K1)], semW)
                pltpu.async_copy(rows2.at[1 - p],
                                 g2_hbm.at[pl.ds((qid - 1) * K2, K2)], semW)

            pltpu.async_copy(ptspad.at[idx1sel.at[p]], rows1.at[p], semG)
            pltpu.async_copy(ptspad.at[idx2sel.at[p]], rows2.at[p], semG)
            return carry

        lax.fori_loop(0, QPW, per_query, jnp.int32(0))

        pltpu.make_async_copy(ptspad.at[pl.ds(0, K1)], rows1.at[0], semG).wait()
        pltpu.make_async_copy(ptspad.at[pl.ds(0, K2)], rows2.at[0], semG).wait()
        pltpu.make_async_copy(ptspad.at[pl.ds(0, K1)], rows1.at[0], semW).wait()
        pltpu.make_async_copy(ptspad.at[pl.ds(0, K2)], rows2.at[0], semW).wait()
        qlast = qbase + QPW - 1
        pltpu.sync_copy(rows1.at[(QPW - 1) % 2],
                        g1_hbm.at[pl.ds(qlast * K1, K1)])
        pltpu.sync_copy(rows2.at[(QPW - 1) % 2],
                        g2_hbm.at[pl.ds(qlast * K2, K2)])

    return sc_ballquery


_SC_CACHE = []


def _sc_ballquery(*args):
    if not _SC_CACHE:
        _SC_CACHE.append(_make_sc_ballquery())
    return _SC_CACHE[0](*args)


def _mlp_body(g1_ref, g2_ref, q1_ref, q2_ref, w1a_ref, b1a_ref, w1b_ref,
              b1b_ref, w2a_ref, b2a_ref, w2b_ref, b2b_ref, w3a_ref, w3b_ref,
              b3_ref, out_ref):
    qblk = out_ref.shape[0]
    r_i = lax.broadcasted_iota(jnp.int32, (128, 128), 0)
    c_i = lax.broadcasted_iota(jnp.int32, (128, 128), 1)
    shift = (((c_i // 16) == (r_i // 16))
             & ((c_i % 16) == (r_i % 16) + 3)
             & ((r_i % 16) < 3)).astype(jnp.float32)

    def branch(g_ref, q_ref, k, wa_ref, ba_ref, wb_ref, bb_ref):
        g = g_ref[...]
        f = (g - q_ref[...]) + jnp.dot(g, shift, precision="highest")
        h = jnp.dot(f, wa_ref[...], precision="highest") + ba_ref[0:1, :]
        h = jnp.maximum(h, 0.0)
        h = jnp.dot(h, wb_ref[...], precision="highest") + bb_ref[0:1, :]
        h = jnp.maximum(h, 0.0)
        h = jnp.max(h.reshape(qblk, k // 8, 256), axis=1)
        m = jnp.maximum(h[:, :128], h[:, 128:])
        m = jnp.maximum(m, pltpu.roll(m, 64, 1))
        m = jnp.maximum(m, pltpu.roll(m, 32, 1))
        return m[:, :32]

    f1 = branch(g1_ref, q1_ref, K1, w1a_ref, b1a_ref, w1b_ref, b1b_ref)
    f2 = branch(g2_ref, q2_ref, K2, w2a_ref, b2a_ref, w2b_ref, b2b_ref)
    out = (jnp.dot(f1, w3a_ref[...], precision="highest")
           + jnp.dot(f2, w3b_ref[...], precision="highest") + b3_ref[0:1, :])
    out_ref[...] = out


def _run_mlp(g1p, g2p, q1p, q2p, w1a, b1a, w1b, b1b, w2a, b2a, w2b, b2b,
             w3a, w3b, b3):
    qblk = 128
    grid = (S // qblk,)

    def full(a):
        return pl.BlockSpec(a.shape, lambda i: (0, 0))

    return pl.pallas_call(
        _mlp_body,
        grid=grid,
        in_specs=[
            pl.BlockSpec((qblk * K1 // 8, 128), lambda i: (i, 0)),
            pl.BlockSpec((qblk * K2 // 8, 128), lambda i: (i, 0)),
            pl.BlockSpec((qblk * K1 // 8, 128), lambda i: (i, 0)),
            pl.BlockSpec((qblk * K2 // 8, 128), lambda i: (i, 0)),
            full(w1a), full(b1a), full(w1b), full(b1b),
            full(w2a), full(b2a), full(w2b), full(b2b),
            full(w3a), full(w3b), full(b3),
        ],
        out_specs=pl.BlockSpec((qblk, 128), lambda i: (i, 0)),
        out_shape=jax.ShapeDtypeStruct((S, 128), jnp.float32),
    )(g1p, g2p, q1p, q2p, w1a, b1a, w1b, b1b, w2a, b2a, w2b, b2b,
      w3a, w3b, b3)


def kernel(pts1, W1a, g1a, b1a, W1b, g1b, b1b, W2a, g2a, b2a, W2b, g2b, b2b,
           W3, b3):
    b = pts1.shape[0]
    p = pts1[0]
    grid = p.reshape(256, 256, 3)
    pts2 = grid[4::8, 4::8].reshape(-1, 3)

    def bf16_rne(x):
        u = lax.bitcast_convert_type(x, jnp.uint32)
        r = (u + jnp.uint32(0x7FFF) + ((u >> 16) & jnp.uint32(1)))
        r = r & jnp.uint32(0xFFFF0000)
        return lax.bitcast_convert_type(r, jnp.float32)

    pbf = bf16_rne(p)
    x2 = jnp.sum(p * p, axis=-1)
    qbf = bf16_rne(pts2)
    q2 = jnp.sum(pts2 * pts2, axis=-1)
    ptspad = jnp.pad(p, ((0, 0), (0, 13)))
    qpad = jnp.pad(pts2, ((0, 0), (0, 13)))

    g1, g2 = _sc_ballquery(pbf[:, 0], pbf[:, 1], pbf[:, 2], x2, ptspad,
                           qbf[:, 0], qbf[:, 1], qbf[:, 2], q2)
    g1p = g1.reshape(S * K1 // 8, 128)
    g2p = g2.reshape(S * K2 // 8, 128)
    q1p = jnp.broadcast_to(qpad[:, None, :], (S, K1, 16)).reshape(-1, 128)
    q2p = jnp.broadcast_to(qpad[:, None, :], (S, K2, 16)).reshape(-1, 128)

    scale = 1.0 / jnp.sqrt(1.0 + EPS)
    eye8 = jnp.eye(8, dtype=jnp.float32)

    def tile8(bias):
        return jnp.tile(bias[None, :], (8, 8 * 16 // bias.shape[0]))

    def fold(w, g, bias, rows):
        wf = w * (g * scale)[None, :]
        wpad = jnp.zeros((16, w.shape[1]), w.dtype).at[:rows].set(wf)
        return jnp.kron(eye8, wpad), tile8(bias)

    w1a_p, b1a_p = fold(W1a, g1a, b1a, 6)
    w2a_p, b2a_p = fold(W2a, g2a, b2a, 6)
    w1b_f = jnp.kron(eye8, W1b * (g1b * scale)[None, :])
    w2b_f = jnp.kron(eye8, W2b * (g2b * scale)[None, :])

    def tile8b(bias):
        return jnp.tile(bias[None, :], (8, 8))

    out = _run_mlp(g1p, g2p, q1p, q2p,
                   w1a_p, b1a_p, w1b_f, tile8b(b1b),
                   w2a_p, b2a_p, w2b_f, tile8b(b2b),
                   W3[:32], W3[32:], jnp.tile(b3[None, :], (8, 1)))
    return out.reshape(b, S, 128)
